# baseline (speedup 1.0000x reference)
import jax
import jax.numpy as jnp
from jax.experimental import pallas as pl

_TAU = 0.5
_ROWS = 4


def _body(logits_ref, uniform_ref, out_ref):
    l = logits_ref[:, 0, :]
    m = jnp.max(l, axis=-1, keepdims=True)
    e = jnp.exp((1.0 / _TAU) * (l - m))
    u = uniform_ref[...]
    r = 1.0 / jnp.log(u)
    r2 = r * r
    z = e[:, None, :] * r2
    s = jnp.sum(z, axis=-1, keepdims=True)
    out_ref[:, 0, :] = jnp.max(z * (1.0 / s), axis=1)


@jax.jit
def kernel(logits, uniform):
    b, d = logits.shape
    _, k, _ = uniform.shape
    rows = _ROWS
    grid = (b // rows,)
    logits3 = logits.reshape(b, 1, d)
    out = pl.pallas_call(
        _body,
        grid=grid,
        in_specs=[
            pl.BlockSpec((rows, 1, d), lambda i: (i, 0, 0)),
            pl.BlockSpec((rows, k, d), lambda i: (i, 0, 0)),
        ],
        out_specs=pl.BlockSpec((rows, 1, d), lambda i: (i, 0, 0)),
        out_shape=jax.ShapeDtypeStruct((b, 1, d), jnp.float32),
    )(logits3, uniform)
    return out.reshape(b, d)


# speedup vs baseline: 1.3305x; 1.3305x over previous
import jax
import jax.numpy as jnp
from jax.experimental import pallas as pl
from jax.experimental.pallas import tpu as pltpu

_TAU = 0.5
_ROWS = 8
_CH = 512


def _body(logits_ref, uniform_ref, out_ref, r2_ref):
    rows, k, d = uniform_ref.shape
    l = logits_ref[...]
    m = jnp.max(l, axis=-1, keepdims=True)
    nc = d // _CH

    s = jnp.zeros((rows, k), jnp.float32)
    for c in range(nc):
        sl = slice(c * _CH, (c + 1) * _CH)
        e_c = jnp.exp((1.0 / _TAU) * (l[:, sl] - m))
        r = 1.0 / jnp.log(uniform_ref[:, :, sl])
        r2 = r * r
        r2_ref[:, :, sl] = r2
        s = s + jnp.sum(r2 * e_c[:, None, :], axis=-1)

    inv = 1.0 / s

    for c in range(nc):
        sl = slice(c * _CH, (c + 1) * _CH)
        e_c = jnp.exp((1.0 / _TAU) * (l[:, sl] - m))
        y = r2_ref[:, :, sl] * inv[:, :, None]
        out_ref[:, sl] = e_c * jnp.max(y, axis=1)


@jax.jit
def kernel(logits, uniform):
    b, d = logits.shape
    _, k, _ = uniform.shape
    rows = _ROWS
    grid = (b // rows,)
    return pl.pallas_call(
        _body,
        grid=grid,
        in_specs=[
            pl.BlockSpec((rows, d), lambda i: (i, 0)),
            pl.BlockSpec((rows, k, d), lambda i: (i, 0, 0)),
        ],
        out_specs=pl.BlockSpec((rows, d), lambda i: (i, 0)),
        out_shape=jax.ShapeDtypeStruct((b, d), jnp.float32),
        scratch_shapes=[pltpu.VMEM((rows, k, d), jnp.float32)],
    )(logits, uniform)
